# baseline (speedup 1.0000x reference)
import jax
import jax.numpy as jnp
import numpy as np
from jax.experimental import pallas as pl
from jax.experimental.pallas import tpu as pltpu

_CUTOFF = 5.0
_NB = 20
_F = 128
_NL = 3
_NMOL = 64
_APM = 64
_MPG = 2
_NA = _MPG * _APM
_ZP = 128
_E = _NB * _APM


def _silu(x):
    return x * jax.nn.sigmoid(x)


def _split(a):
    ah = a.astype(jnp.bfloat16)
    al = (a - ah.astype(jnp.float32)).astype(jnp.bfloat16)
    return ah, al


def _dot3s(ah, al, bh, bl):
    f32 = jnp.float32
    return (jnp.dot(ah, bh, preferred_element_type=f32)
            + (jnp.dot(ah, bl, preferred_element_type=f32)
               + jnp.dot(al, bh, preferred_element_type=f32)))


def _dot3(a, bh, bl):
    ah, al = _split(a)
    return _dot3s(ah, al, bh, bl)


def _geometry(pa, pr):
    f32 = jnp.float32
    vx = pa[:, 0:1] - pr[0:1, :]
    vy = pa[:, 1:2] - pr[1:2, :]
    vz = pa[:, 2:3] - pr[2:3, :]
    d2 = vx * vx + vy * vy + vz * vz
    d = jnp.sqrt(d2 + 1e-12)
    mask = jnp.logical_and(d2 < _CUTOFF * _CUTOFF, d2 > 1e-6).astype(f32)
    fc = 0.5 * (jnp.cos(jnp.pi * d / _CUTOFF) + 1.0) * mask
    inv_d = 1.0 / d
    rx = vx * inv_d
    ry = vy * inv_d
    rz = vz * inv_d

    def tile_lane(a):
        return jnp.concatenate([a] * _NB, axis=1)

    lane = jax.lax.broadcasted_iota(jnp.int32, (_APM, _E), 1)
    nb = (lane // _APM + 1).astype(f32)
    dt = tile_lane(d)
    base = (tile_lane(fc) * jnp.sin(nb * (np.pi / _CUTOFF) * dt) / dt
            * np.sqrt(2.0 / _CUTOFF))
    SK = jnp.concatenate(
        [base * tile_lane(rx), base * tile_lane(ry), base * tile_lane(rz)],
        axis=0)
    return _split(base), _split(SK)


def _painn_body(atz_ref, posc_ref, posr_ref, table_ref, *refs):
    out_ref = refs[-1]
    wref = refs[:-1]
    f32 = jnp.float32
    half = _APM

    z = atz_ref[...].reshape(_NA, 1)
    zio = jax.lax.broadcasted_iota(jnp.int32, (_NA, _ZP), 1)
    oneh = (z == zio).astype(f32)
    t0 = jnp.dot(oneh, table_ref[...], preferred_element_type=f32,
                 precision=jax.lax.Precision.HIGHEST)
    xs = t0[:, :_F]
    e_sp = [jnp.sum(t0[m * half:(m + 1) * half, _F:_F + 1])
            for m in range(_MPG)]

    geo = [_geometry(posc_ref[m], posr_ref[m]) for m in range(_MPG)]

    xvx = jnp.zeros((_NA, _F), f32)
    xvy = jnp.zeros((_NA, _F), f32)
    xvz = jnp.zeros((_NA, _F), f32)

    def rep_row(a):
        return jnp.concatenate([a] * _NB, axis=0)

    idx = 0
    for _ in range(_NL):
        (Wm1h, Wm1l, bm1, Wm2h, Wm2l, bm2, Wrexp,
         WUh, WUl, WVh, WVl,
         Wu1h, Wu1l, bu1, Wu2h, Wu2l, bu2) = wref[idx:idx + 17]
        idx += 17
        phi = _dot3(
            _silu(_dot3(xs, Wm1h[...], Wm1l[...]) + bm1[...]),
            Wm2h[...], Wm2l[...]) + bm2[...]
        wr = Wrexp[...]
        ds_parts, dvx_parts, dvy_parts, dvz_parts = [], [], [], []
        for m in range(_MPG):
            sl = slice(m * half, (m + 1) * half)
            phim = phi[sl]
            p2 = phim[:, _F:2 * _F]
            G1 = wr[:, :_F] * rep_row(phim[:, :_F])
            G2x = wr[:, _F:2 * _F] * rep_row(p2 * xvx[sl])
            G2y = wr[:, _F:2 * _F] * rep_row(p2 * xvy[sl])
            G2z = wr[:, _F:2 * _F] * rep_row(p2 * xvz[sl])
            G3 = wr[:, 2 * _F:] * rep_row(phim[:, 2 * _F:])
            rhs = jnp.concatenate([G1, G2x, G2y, G2z], axis=1)
            rh, rl = _split(rhs)
            g3h, g3l = _split(G3)
            (sh, sl2), (skh, skl) = geo[m]
            big = _dot3s(sh, sl2, rh, rl)
            dvs = _dot3s(skh, skl, g3h, g3l)
            ds_parts.append(big[:, :_F])
            dvx_parts.append(big[:, _F:2 * _F] + dvs[:half])
            dvy_parts.append(big[:, 2 * _F:3 * _F] + dvs[half:2 * half])
            dvz_parts.append(big[:, 3 * _F:] + dvs[2 * half:])
        xs = xs + jnp.concatenate(ds_parts, axis=0)
        xvx = xvx + jnp.concatenate(dvx_parts, axis=0)
        xvy = xvy + jnp.concatenate(dvy_parts, axis=0)
        xvz = xvz + jnp.concatenate(dvz_parts, axis=0)
        xv_all = jnp.concatenate([xvx, xvy, xvz], axis=0)
        xh, xl = _split(xv_all)
        U = _dot3s(xh, xl, WUh[...], WUl[...])
        Vt = _dot3s(xh, xl, WVh[...], WVl[...])
        Ux, Uy, Uz = U[:_NA], U[_NA:2 * _NA], U[2 * _NA:]
        Vx, Vy, Vz = Vt[:_NA], Vt[_NA:2 * _NA], Vt[2 * _NA:]
        Vn = jnp.sqrt(Vx * Vx + Vy * Vy + Vz * Vz + 1e-8)
        cat = jnp.concatenate([xs, Vn], axis=1)
        a = _dot3(
            _silu(_dot3(cat, Wu1h[...], Wu1l[...]) + bu1[...]),
            Wu2h[...], Wu2l[...]) + bu2[...]
        a_vv = a[:, 2 * _F:]
        xs = xs + a[:, :_F] + a[:, _F:2 * _F] * (Ux * Vx + Uy * Vy + Uz * Vz)
        xvx = xvx + a_vv * Ux
        xvy = xvy + a_vv * Uy
        xvz = xvz + a_vv * Uz

    Wo1h, Wo1l, bo1, Wo2h, Wo2l, bo2b = wref[idx:idx + 6]
    h = _dot3(
        _silu(_dot3(xs, Wo1h[...], Wo1l[...]) + bo1[...]),
        Wo2h[...], Wo2l[...]) + bo2b[...]
    outs = [jnp.full((1, 1, _F),
                     jnp.sum(h[m * half:(m + 1) * half, 0:1]) + e_sp[m], f32)
            for m in range(_MPG)]
    out_ref[...] = jnp.concatenate(outs, axis=0)


def kernel(at_no, pos, batch, params):
    del batch
    f32 = jnp.float32
    pos = (pos * 1.0).astype(f32)
    atz = at_no.astype(jnp.int32).reshape(_NMOL, _APM, 1)
    posc = pos.reshape(_NMOL, _APM, 3)
    posr = jnp.transpose(posc, (0, 2, 1))

    maxz = params['emb'].shape[0]
    table = jnp.zeros((_ZP, 2 * _F), f32)
    table = table.at[:maxz, :_F].set(params['emb'].astype(f32))
    table = table.at[:maxz, _F].set(params['atom_sp'].astype(f32))

    def hl(w):
        return _split(w.astype(f32))

    wlist = []
    for p in params['layers']:
        wlist += [
            *hl(p['Wm1']), p['bm1'].reshape(1, _F).astype(f32),
            *hl(p['Wm2']), p['bm2'].reshape(1, 3 * _F).astype(f32),
            jnp.repeat(p['Wrbf'].astype(f32), _APM, axis=0),
            *hl(p['WU']), *hl(p['WV']),
            *hl(p['Wu1']), p['bu1'].reshape(1, _F).astype(f32),
            *hl(p['Wu2']), p['bu2'].reshape(1, 3 * _F).astype(f32),
        ]
    halfF = _F // 2
    wo2p = jnp.zeros((halfF, _F), f32).at[:, 0].set(params['Wo2'][:, 0].astype(f32))
    bo2b = jnp.broadcast_to(params['bo2'].reshape(1, 1).astype(f32), (1, _F))
    wlist += [*hl(params['Wo1'].astype(f32)),
              params['bo1'].reshape(1, halfF).astype(f32),
              *hl(wo2p), bo2b]

    in_specs = [
        pl.BlockSpec((_MPG, _APM, 1), lambda m: (m, 0, 0)),
        pl.BlockSpec((_MPG, _APM, 3), lambda m: (m, 0, 0)),
        pl.BlockSpec((_MPG, 3, _APM), lambda m: (m, 0, 0)),
        pl.BlockSpec(table.shape, lambda m: (0, 0)),
    ] + [pl.BlockSpec(w.shape, lambda m: (0, 0)) for w in wlist]

    out = pl.pallas_call(
        _painn_body,
        grid=(_NMOL // _MPG,),
        in_specs=in_specs,
        out_specs=pl.BlockSpec((_MPG, 1, _F), lambda m: (m, 0, 0)),
        out_shape=jax.ShapeDtypeStruct((_NMOL, 1, _F), f32),
        compiler_params=pltpu.CompilerParams(
            dimension_semantics=("arbitrary",)),
    )(atz, posc, posr, table, *wlist)
    return out[:, 0, 0]


# speedup vs baseline: 1.4091x; 1.4091x over previous
import jax
import jax.numpy as jnp
import numpy as np
from jax.experimental import pallas as pl
from jax.experimental.pallas import tpu as pltpu

_CUTOFF = 5.0
_NB = 20
_F = 128
_NL = 3
_NMOL = 64
_APM = 64
_ZP = 128
_E = _NB * _APM


def _silu(x):
    return x * jax.nn.sigmoid(x)


def _split(a):
    ah = a.astype(jnp.bfloat16)
    al = (a - ah.astype(jnp.float32)).astype(jnp.bfloat16)
    return ah, al


def _dot3s(ah, al, bh, bl):
    f32 = jnp.float32
    return (jnp.dot(ah, bh, preferred_element_type=f32)
            + (jnp.dot(ah, bl, preferred_element_type=f32)
               + jnp.dot(al, bh, preferred_element_type=f32)))


def _dot3(a, bh, bl):
    ah, al = _split(a)
    return _dot3s(ah, al, bh, bl)


def _painn_body(atz_ref, posc_ref, posr_ref, tabh_ref, tabl_ref,
                m1_ref, bm1_ref, m2_ref, bm2_ref, wrb_ref,
                wu_ref, wv_ref, u1_ref, bu1_ref, u2_ref, bu2_ref,
                o1_ref, bo1_ref, o2_ref, bo2_ref,
                out_ref,
                m1h, m1l, m2h, m2l, wuh, wul, wvh, wvl,
                u1h, u1l, u2h, u2l, o1h, o1l, o2h, o2l, wrx):
    f32 = jnp.float32

    @pl.when(pl.program_id(0) == 0)
    def _prep():
        for src, dh, dl in ((m1_ref, m1h, m1l), (m2_ref, m2h, m2l),
                            (wu_ref, wuh, wul), (wv_ref, wvh, wvl),
                            (u1_ref, u1h, u1l), (u2_ref, u2h, u2l),
                            (o1_ref, o1h, o1l), (o2_ref, o2h, o2l)):
            h, l = _split(src[...])
            dh[...] = h
            dl[...] = l
        wr = wrb_ref[...]
        for li in range(_NL):
            for b in range(_NB):
                wrx[li, b * _APM:(b + 1) * _APM, :] = jnp.broadcast_to(
                    wr[li, b:b + 1, :], (_APM, 3 * _F))

    z = atz_ref[0]
    zio = jax.lax.broadcasted_iota(jnp.int32, (_APM, _ZP), 1)
    oneh = (z == zio).astype(jnp.bfloat16)
    t0 = (jnp.dot(oneh, tabh_ref[...], preferred_element_type=f32)
          + jnp.dot(oneh, tabl_ref[...], preferred_element_type=f32))
    xs = t0[:, :_F]
    e_sp = jnp.sum(t0[:, _F:_F + 1])

    pa = posc_ref[0]
    pr = posr_ref[0]
    vx = pa[:, 0:1] - pr[0:1, :]
    vy = pa[:, 1:2] - pr[1:2, :]
    vz = pa[:, 2:3] - pr[2:3, :]
    d2 = vx * vx + vy * vy + vz * vz
    d = jnp.sqrt(d2 + 1e-12)
    mask = jnp.logical_and(d2 < _CUTOFF * _CUTOFF, d2 > 1e-6).astype(f32)
    fc = 0.5 * (jnp.cos(jnp.pi * d / _CUTOFF) + 1.0) * mask
    inv_d = 1.0 / d
    rx = vx * inv_d
    ry = vy * inv_d
    rz = vz * inv_d

    def tile_lane(a):
        return jnp.concatenate([a] * _NB, axis=1)

    def rep_row(a):
        return jnp.concatenate([a] * _NB, axis=0)

    lane = jax.lax.broadcasted_iota(jnp.int32, (_APM, _E), 1)
    nb = (lane // _APM + 1).astype(f32)
    dt = tile_lane(d)
    base = (tile_lane(fc) * jnp.sin(nb * (np.pi / _CUTOFF) * dt) / dt
            * np.sqrt(2.0 / _CUTOFF))
    Sh, Sl = _split(base)
    SK = jnp.concatenate(
        [base * tile_lane(rx), base * tile_lane(ry), base * tile_lane(rz)],
        axis=0)
    SKh, SKl = _split(SK)

    xvx = jnp.zeros((_APM, _F), f32)
    xvy = jnp.zeros((_APM, _F), f32)
    xvz = jnp.zeros((_APM, _F), f32)

    for li in range(_NL):
        phi = _dot3(
            _silu(_dot3(xs, m1h[li], m1l[li]) + bm1_ref[li]),
            m2h[li], m2l[li]) + bm2_ref[li]
        wr = wrx[li]
        G3 = wr[:, 2 * _F:] * rep_row(phi[:, 2 * _F:])
        g3h, g3l = _split(G3)
        dvs = _dot3s(SKh, SKl, g3h, g3l)
        if li == 0:
            G1 = wr[:, :_F] * rep_row(phi[:, :_F])
            rh, rl = _split(G1)
            big = _dot3s(Sh, Sl, rh, rl)
            xs = xs + big
            xvx = dvs[:_APM]
            xvy = dvs[_APM:2 * _APM]
            xvz = dvs[2 * _APM:]
        else:
            p2 = phi[:, _F:2 * _F]
            G1 = wr[:, :_F] * rep_row(phi[:, :_F])
            G2x = wr[:, _F:2 * _F] * rep_row(p2 * xvx)
            G2y = wr[:, _F:2 * _F] * rep_row(p2 * xvy)
            G2z = wr[:, _F:2 * _F] * rep_row(p2 * xvz)
            rhs = jnp.concatenate([G1, G2x, G2y, G2z], axis=1)
            rh, rl = _split(rhs)
            big = _dot3s(Sh, Sl, rh, rl)
            xs = xs + big[:, :_F]
            xvx = xvx + big[:, _F:2 * _F] + dvs[:_APM]
            xvy = xvy + big[:, 2 * _F:3 * _F] + dvs[_APM:2 * _APM]
            xvz = xvz + big[:, 3 * _F:] + dvs[2 * _APM:]
        xv_all = jnp.concatenate([xvx, xvy, xvz], axis=0)
        xh, xl = _split(xv_all)
        U = _dot3s(xh, xl, wuh[li], wul[li])
        Vt = _dot3s(xh, xl, wvh[li], wvl[li])
        Ux, Uy, Uz = U[:_APM], U[_APM:2 * _APM], U[2 * _APM:]
        Vx, Vy, Vz = Vt[:_APM], Vt[_APM:2 * _APM], Vt[2 * _APM:]
        Vn = jnp.sqrt(Vx * Vx + Vy * Vy + Vz * Vz + 1e-8)
        cat = jnp.concatenate([xs, Vn], axis=1)
        a = _dot3(
            _silu(_dot3(cat, u1h[li], u1l[li]) + bu1_ref[li]),
            u2h[li], u2l[li]) + bu2_ref[li]
        a_vv = a[:, 2 * _F:]
        xs = xs + a[:, :_F] + a[:, _F:2 * _F] * (Ux * Vx + Uy * Vy + Uz * Vz)
        xvx = xvx + a_vv * Ux
        xvy = xvy + a_vv * Uy
        xvz = xvz + a_vv * Uz

    h = _dot3(
        _silu(_dot3(xs, o1h[...], o1l[...]) + bo1_ref[...]),
        o2h[...], o2l[...]) + bo2_ref[...]
    e = jnp.sum(h[:, 0:1]) + e_sp
    out_ref[...] = jnp.full((1, 1, _F), e, f32)


def kernel(at_no, pos, batch, params):
    del batch
    f32 = jnp.float32
    bf16 = jnp.bfloat16
    pos = (pos * 1.0).astype(f32)
    atz = at_no.astype(jnp.int32).reshape(_NMOL, _APM, 1)
    posc = pos.reshape(_NMOL, _APM, 3)
    posr = jnp.transpose(posc, (0, 2, 1))

    maxz = params['emb'].shape[0]
    table = jnp.zeros((_ZP, 2 * _F), f32)
    table = table.at[:maxz, :_F].set(params['emb'].astype(f32))
    table = table.at[:maxz, _F].set(params['atom_sp'].astype(f32))
    tabh, tabl = _split(table)

    L = params['layers']

    def stk(name, shp):
        return jnp.stack([p[name].astype(f32).reshape(shp) for p in L])

    m1 = stk('Wm1', (_F, _F))
    bm1 = stk('bm1', (1, _F))
    m2 = stk('Wm2', (_F, 3 * _F))
    bm2 = stk('bm2', (1, 3 * _F))
    wrb = stk('Wrbf', (_NB, 3 * _F))
    wu = stk('WU', (_F, _F))
    wv = stk('WV', (_F, _F))
    u1 = stk('Wu1', (2 * _F, _F))
    bu1 = stk('bu1', (1, _F))
    u2 = stk('Wu2', (_F, 3 * _F))
    bu2 = stk('bu2', (1, 3 * _F))
    half = _F // 2
    o1 = params['Wo1'].astype(f32)
    bo1 = params['bo1'].reshape(1, half).astype(f32)
    o2 = jnp.zeros((half, _F), f32).at[:, 0].set(params['Wo2'][:, 0].astype(f32))
    bo2 = jnp.broadcast_to(params['bo2'].reshape(1, 1).astype(f32), (1, _F))

    warrs = [m1, bm1, m2, bm2, wrb, wu, wv, u1, bu1, u2, bu2, o1, bo1, o2, bo2]

    def wspec(a):
        n = a.ndim
        if n == 3:
            return pl.BlockSpec(a.shape, lambda m: (0, 0, 0))
        return pl.BlockSpec(a.shape, lambda m: (0, 0))

    in_specs = [
        pl.BlockSpec((1, _APM, 1), lambda m: (m, 0, 0)),
        pl.BlockSpec((1, _APM, 3), lambda m: (m, 0, 0)),
        pl.BlockSpec((1, 3, _APM), lambda m: (m, 0, 0)),
        pl.BlockSpec(table.shape, lambda m: (0, 0)),
        pl.BlockSpec(table.shape, lambda m: (0, 0)),
    ] + [wspec(w) for w in warrs]

    scratch = [
        pltpu.VMEM((_NL, _F, _F), bf16), pltpu.VMEM((_NL, _F, _F), bf16),
        pltpu.VMEM((_NL, _F, 3 * _F), bf16), pltpu.VMEM((_NL, _F, 3 * _F), bf16),
        pltpu.VMEM((_NL, _F, _F), bf16), pltpu.VMEM((_NL, _F, _F), bf16),
        pltpu.VMEM((_NL, _F, _F), bf16), pltpu.VMEM((_NL, _F, _F), bf16),
        pltpu.VMEM((_NL, 2 * _F, _F), bf16), pltpu.VMEM((_NL, 2 * _F, _F), bf16),
        pltpu.VMEM((_NL, _F, 3 * _F), bf16), pltpu.VMEM((_NL, _F, 3 * _F), bf16),
        pltpu.VMEM((_F, half), bf16), pltpu.VMEM((_F, half), bf16),
        pltpu.VMEM((half, _F), bf16), pltpu.VMEM((half, _F), bf16),
        pltpu.VMEM((_NL, _E, 3 * _F), f32),
    ]

    out = pl.pallas_call(
        _painn_body,
        grid=(_NMOL,),
        in_specs=in_specs,
        out_specs=pl.BlockSpec((1, 1, _F), lambda m: (m, 0, 0)),
        out_shape=jax.ShapeDtypeStruct((_NMOL, 1, _F), f32),
        scratch_shapes=scratch,
        compiler_params=pltpu.CompilerParams(
            dimension_semantics=("arbitrary",)),
    )(atz, posc, posr, tabh, tabl, *warrs)
    return out[:, 0, 0]
